# baseline (speedup 1.0000x reference)
import functools

import jax
import jax.numpy as jnp
from jax import lax
from jax.experimental import pallas as pl
from jax.experimental.pallas import tpu as pltpu
from jax.experimental.pallas import tpu_sc as plsc

K_NN = 10
GS = 128
QT = 256
KT = 2048
GT = KT // GS
NEG = float("-inf")
BIG = 2**30


def _p1_body(n_keys, ki_grid, qt, q_ref, k_ref, s_ref, m_ref):
    ki = pl.program_id(1)
    scores = lax.dot_general(
        q_ref[...], k_ref[...], (((1,), (1,)), ((), ())),
        preferred_element_type=jnp.float32)
    for j in range(GT):
        s_ref[:, j, :] = scores[:, j * GS:(j + 1) * GS]

    def gmax_of(x):
        return jnp.concatenate(
            [jnp.max(x[:, j * GS:(j + 1) * GS], axis=1, keepdims=True)
             for j in range(GT)], axis=1)

    full_ki = n_keys // KT
    for c in range(8):
        @pl.when((ki % 8 == c) & (ki < full_ki))
        def _():
            m_ref[:, c * GT:(c + 1) * GT] = gmax_of(scores)
    n_rem = n_keys % KT
    if n_rem:
        c_p = full_ki % 8
        @pl.when(ki == full_ki)
        def _():
            col_iota = lax.broadcasted_iota(jnp.int32, (qt, KT), 1)
            masked = jnp.where(col_iota < n_rem, scores, NEG)
            m_ref[:, c_p * GT:(c_p + 1) * GT] = gmax_of(masked)
    start = ((ki_grid - 1) % 8 + 1) * GT
    if start < 128:
        @pl.when(ki == ki_grid - 1)
        def _():
            m_ref[:, start:] = jnp.full((qt, 128 - start), NEG, jnp.float32)


def _p1b_body(n_groups, m_ref, r_ref):
    qi = pl.program_id(0)
    m = m_ref[...]
    mw = m.shape[1]
    gids = lax.broadcasted_iota(jnp.int32, (QT, mw), 1)
    qvec = qi * QT + lax.broadcasted_iota(jnp.int32, (QT, 1), 0)
    picks = []
    for _ in range(K_NN):
        mx = jnp.max(m, axis=1, keepdims=True)
        g = jnp.min(jnp.where(m == mx, gids, BIG), axis=1, keepdims=True)
        m = jnp.where(gids == g, NEG, m)
        picks.append(qvec * n_groups + g)
    picks.extend([picks[-1]] * (16 - K_NN))
    r_ref[...] = jnp.concatenate(picks, axis=1).astype(jnp.int32)


def _sc_gather_body(chunks, table_hbm, idx_hbm, out_hbm, idx_v, rows_v, sem):
    wid = lax.axis_index("s") * 2 + lax.axis_index("c")
    pltpu.sync_copy(idx_hbm.at[pl.ds(wid * chunks, chunks)], idx_v)
    for c in range(chunks):
        pltpu.async_copy(table_hbm.at[idx_v.at[c]], rows_v, sem).wait()
        pltpu.sync_copy(rows_v,
                        out_hbm.at[pl.ds((wid * chunks + c) * 128, 128)])


def _p3_body(n_keys, n_groups, c_ref, r_ref, s_out, i_out):
    qi = pl.program_id(0)
    cand = jnp.concatenate([c_ref[:, j, :] for j in range(16)], axis=1)
    qvec = qi * QT + lax.broadcasted_iota(jnp.int32, (QT, 1), 0)
    g = r_ref[...] - qvec * n_groups
    lane = lax.broadcasted_iota(jnp.int32, (QT, GS), 1)
    idx = jnp.concatenate(
        [g[:, j:j + 1] * GS + lane for j in range(16)], axis=1)
    col = lax.broadcasted_iota(jnp.int32, (QT, 16 * GS), 1)
    cand = jnp.where((idx < n_keys) & (col < K_NN * GS), cand, NEG)
    svals, ivals = [], []
    for _ in range(K_NN):
        mx = jnp.max(cand, axis=1, keepdims=True)
        best = jnp.min(jnp.where(cand == mx, idx, BIG), axis=1,
                       keepdims=True)
        cand = jnp.where(idx == best, NEG, cand)
        svals.append(mx)
        ivals.append(best)
    s_out[...] = jnp.concatenate(svals, axis=1)
    i_out[...] = jnp.concatenate(ivals, axis=1).astype(jnp.int32)


def kernel(queries, keys):
    nq, d = queries.shape
    n_keys = keys.shape[0]
    ki_grid = -(-n_keys // KT)
    kp = ki_grid * KT
    n_groups = kp // GS
    mw = -(-ki_grid // 8) * 128
    p1_qt = min(2048, nq)
    qi1_grid = nq // p1_qt
    qi_grid = nq // QT

    keys_p = jnp.concatenate(
        [keys, jnp.zeros((kp - n_keys, d), keys.dtype)], axis=0)

    s_full, m = pl.pallas_call(
        functools.partial(_p1_body, n_keys, ki_grid, p1_qt),
        grid=(qi1_grid, ki_grid),
        in_specs=[
            pl.BlockSpec((p1_qt, d), lambda qi, ki: (qi, 0)),
            pl.BlockSpec((KT, d), lambda qi, ki: (ki, 0)),
        ],
        out_specs=[
            pl.BlockSpec((p1_qt, GT, GS), lambda qi, ki: (qi, ki, 0)),
            pl.BlockSpec((p1_qt, 128), lambda qi, ki: (qi, ki // 8)),
        ],
        out_shape=[
            jax.ShapeDtypeStruct((nq, n_groups, GS), jnp.float32),
            jax.ShapeDtypeStruct((nq, mw), jnp.float32),
        ],
    )(queries, keys_p)

    rowids = pl.pallas_call(
        functools.partial(_p1b_body, n_groups),
        grid=(qi_grid,),
        in_specs=[pl.BlockSpec((QT, mw), lambda qi: (qi, 0))],
        out_specs=pl.BlockSpec((QT, 16), lambda qi: (qi, 0)),
        out_shape=jax.ShapeDtypeStruct((nq, 16), jnp.int32),
    )(m)

    n_rows = nq * 16
    chunks = n_rows // (32 * 128)
    mesh = plsc.VectorSubcoreMesh(core_axis_name="c", subcore_axis_name="s")
    gathered = pl.kernel(
        functools.partial(_sc_gather_body, chunks),
        mesh=mesh,
        out_type=jax.ShapeDtypeStruct((n_rows, GS), jnp.float32),
        scratch_types=[
            pltpu.VMEM((chunks, 128), jnp.int32),
            pltpu.VMEM((128, GS), jnp.float32),
            pltpu.SemaphoreType.DMA,
        ],
    )(s_full.reshape(nq * n_groups, GS),
      rowids.reshape(n_rows // 128, 128))

    return pl.pallas_call(
        functools.partial(_p3_body, n_keys, n_groups),
        grid=(qi_grid,),
        in_specs=[
            pl.BlockSpec((QT, 16, GS), lambda qi: (qi, 0, 0)),
            pl.BlockSpec((QT, 16), lambda qi: (qi, 0)),
        ],
        out_specs=[
            pl.BlockSpec((QT, K_NN), lambda qi: (qi, 0)),
            pl.BlockSpec((QT, K_NN), lambda qi: (qi, 0)),
        ],
        out_shape=[
            jax.ShapeDtypeStruct((nq, K_NN), jnp.float32),
            jax.ShapeDtypeStruct((nq, K_NN), jnp.int32),
        ],
    )(gathered.reshape(nq, 16, GS), rowids)


# speedup vs baseline: 2.2531x; 2.2531x over previous
import functools

import jax
import jax.numpy as jnp
from jax import lax
from jax.experimental import pallas as pl
from jax.experimental.pallas import tpu as pltpu
from jax.experimental.pallas import tpu_sc as plsc

K_NN = 10
GS = 128
QT = 256
KT = 2048
GT = KT // GS
NEG = float("-inf")
BIG = 2**30


def _p1_body(n_keys, ki_grid, qt, q_ref, k_ref, s_ref, m_ref):
    ki = pl.program_id(1)
    scores = lax.dot_general(
        q_ref[...], k_ref[...], (((1,), (1,)), ((), ())),
        preferred_element_type=jnp.float32)
    for j in range(GT):
        s_ref[:, j, :] = scores[:, j * GS:(j + 1) * GS]

    def gmax_of(x):
        return jnp.concatenate(
            [jnp.max(x[:, j * GS:(j + 1) * GS], axis=1, keepdims=True)
             for j in range(GT)], axis=1)

    full_ki = n_keys // KT
    for c in range(8):
        @pl.when((ki % 8 == c) & (ki < full_ki))
        def _():
            m_ref[:, c * GT:(c + 1) * GT] = gmax_of(scores)
    n_rem = n_keys % KT
    if n_rem:
        c_p = full_ki % 8
        @pl.when(ki == full_ki)
        def _():
            col_iota = lax.broadcasted_iota(jnp.int32, (qt, KT), 1)
            masked = jnp.where(col_iota < n_rem, scores, NEG)
            m_ref[:, c_p * GT:(c_p + 1) * GT] = gmax_of(masked)
    start = ((ki_grid - 1) % 8 + 1) * GT
    if start < 128:
        @pl.when(ki == ki_grid - 1)
        def _():
            m_ref[:, start:] = jnp.full((qt, 128 - start), NEG, jnp.float32)


def _p1b_body(n_groups, m_ref, r_ref):
    qi = pl.program_id(0)
    m = m_ref[...]
    mw = m.shape[1]
    gids = lax.broadcasted_iota(jnp.int32, (QT, mw), 1)
    qvec = qi * QT + lax.broadcasted_iota(jnp.int32, (QT, 1), 0)
    picks = []
    for _ in range(K_NN):
        mx = jnp.max(m, axis=1, keepdims=True)
        g = jnp.min(jnp.where(m == mx, gids, BIG), axis=1, keepdims=True)
        m = jnp.where(gids == g, NEG, m)
        picks.append(qvec * n_groups + g)
    picks.extend([picks[-1]] * (16 - K_NN))
    r_ref[...] = jnp.concatenate(picks, axis=1).astype(jnp.int32)


def _sc_gather_body(chunks, table_hbm, idx_hbm, out_hbm, idx_v, rows_v, sem):
    wid = lax.axis_index("s") * 2 + lax.axis_index("c")
    pltpu.sync_copy(idx_hbm.at[pl.ds(wid * chunks, chunks)], idx_v)
    for c in range(chunks):
        pltpu.async_copy(table_hbm.at[idx_v.at[c]], rows_v, sem).wait()
        pltpu.sync_copy(rows_v,
                        out_hbm.at[pl.ds((wid * chunks + c) * 128, 128)])


def _p3_body(n_keys, n_groups, c_ref, r_ref, s_out, i_out):
    qi = pl.program_id(0)
    cand = jnp.concatenate([c_ref[:, j, :] for j in range(16)], axis=1)
    qvec = qi * QT + lax.broadcasted_iota(jnp.int32, (QT, 1), 0)
    g = r_ref[...] - qvec * n_groups
    lane = lax.broadcasted_iota(jnp.int32, (QT, GS), 1)
    idx = jnp.concatenate(
        [g[:, j:j + 1] * GS + lane for j in range(16)], axis=1)
    col = lax.broadcasted_iota(jnp.int32, (QT, 16 * GS), 1)
    cand = jnp.where((idx < n_keys) & (col < K_NN * GS), cand, NEG)
    svals, ivals = [], []
    for _ in range(K_NN):
        mx = jnp.max(cand, axis=1, keepdims=True)
        best = jnp.min(jnp.where(cand == mx, idx, BIG), axis=1,
                       keepdims=True)
        cand = jnp.where(idx == best, NEG, cand)
        svals.append(mx)
        ivals.append(best)
    s_out[...] = jnp.concatenate(svals, axis=1)
    i_out[...] = jnp.concatenate(ivals, axis=1).astype(jnp.int32)


def kernel(queries, keys):
    nq, d = queries.shape
    n_keys = keys.shape[0]
    ki_grid = -(-n_keys // KT)
    kp = ki_grid * KT
    n_groups = kp // GS
    mw = -(-ki_grid // 8) * 128
    p1_qt = min(256, nq)
    qi1_grid = nq // p1_qt
    qi_grid = nq // QT

    keys_p = jnp.concatenate(
        [keys, jnp.zeros((kp - n_keys, d), keys.dtype)], axis=0)

    s_full, m = pl.pallas_call(
        functools.partial(_p1_body, n_keys, ki_grid, p1_qt),
        grid=(qi1_grid, ki_grid),
        in_specs=[
            pl.BlockSpec((p1_qt, d), lambda qi, ki: (qi, 0)),
            pl.BlockSpec((KT, d), lambda qi, ki: (ki, 0)),
        ],
        out_specs=[
            pl.BlockSpec((p1_qt, GT, GS), lambda qi, ki: (qi, ki, 0)),
            pl.BlockSpec((p1_qt, 128), lambda qi, ki: (qi, ki // 8)),
        ],
        out_shape=[
            jax.ShapeDtypeStruct((nq, n_groups, GS), jnp.float32),
            jax.ShapeDtypeStruct((nq, mw), jnp.float32),
        ],
    )(queries, keys_p)

    rowids = pl.pallas_call(
        functools.partial(_p1b_body, n_groups),
        grid=(qi_grid,),
        in_specs=[pl.BlockSpec((QT, mw), lambda qi: (qi, 0))],
        out_specs=pl.BlockSpec((QT, 16), lambda qi: (qi, 0)),
        out_shape=jax.ShapeDtypeStruct((nq, 16), jnp.int32),
    )(m)

    n_rows = nq * 16
    chunks = n_rows // (32 * 128)
    mesh = plsc.VectorSubcoreMesh(core_axis_name="c", subcore_axis_name="s")
    gathered = pl.kernel(
        functools.partial(_sc_gather_body, chunks),
        mesh=mesh,
        out_type=jax.ShapeDtypeStruct((n_rows, GS), jnp.float32),
        scratch_types=[
            pltpu.VMEM((chunks, 128), jnp.int32),
            pltpu.VMEM((128, GS), jnp.float32),
            pltpu.SemaphoreType.DMA,
        ],
    )(s_full.reshape(nq * n_groups, GS),
      rowids.reshape(n_rows // 128, 128))

    return pl.pallas_call(
        functools.partial(_p3_body, n_keys, n_groups),
        grid=(qi_grid,),
        in_specs=[
            pl.BlockSpec((QT, 16, GS), lambda qi: (qi, 0, 0)),
            pl.BlockSpec((QT, 16), lambda qi: (qi, 0)),
        ],
        out_specs=[
            pl.BlockSpec((QT, K_NN), lambda qi: (qi, 0)),
            pl.BlockSpec((QT, K_NN), lambda qi: (qi, 0)),
        ],
        out_shape=[
            jax.ShapeDtypeStruct((nq, K_NN), jnp.float32),
            jax.ShapeDtypeStruct((nq, K_NN), jnp.int32),
        ],
    )(gathered.reshape(nq, 16, GS), rowids)


# speedup vs baseline: 3.2947x; 1.4623x over previous
import functools

import jax
import jax.numpy as jnp
from jax import lax
from jax.experimental import pallas as pl
from jax.experimental.pallas import tpu as pltpu
from jax.experimental.pallas import tpu_sc as plsc

K_NN = 10
GS = 128
QT = 256
KT = 2048
GT = KT // GS
NEG = float("-inf")
BIG = 2**30


def _p1_body(n_keys, ki_grid, q_ref, k_ref, s_ref, m_ref):
    ki = pl.program_id(1)
    scores = lax.dot_general(
        q_ref[...], k_ref[...], (((1,), (1,)), ((), ())),
        preferred_element_type=jnp.float32)
    for j in range(GT):
        s_ref[j] = scores[:, j * GS:(j + 1) * GS]
    col_iota = lax.broadcasted_iota(jnp.int32, (QT, KT), 1)
    masked = jnp.where(col_iota + ki * KT < n_keys, scores, NEG)
    gmax = jnp.concatenate(
        [jnp.max(masked[:, j * GS:(j + 1) * GS], axis=1, keepdims=True)
         for j in range(GT)], axis=1)
    gmax_t = jnp.transpose(gmax)
    for c in range(8):
        @pl.when(ki % 8 == c)
        def _():
            m_ref[c * GT:(c + 1) * GT, :] = gmax_t
    start = ((ki_grid - 1) % 8 + 1) * GT
    if start < 128:
        @pl.when(ki == ki_grid - 1)
        def _():
            m_ref[start:, :] = jnp.full((128 - start, QT), NEG, jnp.float32)


def _p1b_body(nq, m_ref, r_ref, g_ref):
    qi = pl.program_id(0)
    m = m_ref[...]
    mh = m.shape[0]
    gids = lax.broadcasted_iota(jnp.int32, (mh, QT), 0)
    picks = []
    for _ in range(K_NN):
        mx = jnp.max(m, axis=0, keepdims=True)
        g = jnp.min(jnp.where(m == mx, gids, BIG), axis=0, keepdims=True)
        m = jnp.where(gids == g, NEG, m)
        picks.append(g)
    picks.extend([picks[-1]] * (16 - K_NN))
    gmat = jnp.concatenate(picks, axis=0)
    qlane = qi * QT + lax.broadcasted_iota(jnp.int32, (1, QT), 1)
    r_ref[...] = gmat * nq + qlane
    g_ref[...] = jnp.transpose(gmat)


def _sc_gather_body(chunks, table_hbm, idx_hbm, out_hbm, idx_v, rows_v, sem):
    wid = lax.axis_index("s") * 2 + lax.axis_index("c")
    pltpu.sync_copy(idx_hbm.at[pl.ds(wid * chunks, chunks)], idx_v)
    for c in range(chunks):
        pltpu.async_copy(table_hbm.at[idx_v.at[c]], rows_v, sem).wait()
        pltpu.sync_copy(rows_v,
                        out_hbm.at[pl.ds((wid * chunks + c) * 128, 128)])


def _p3_body(n_keys, c_ref, g_ref, s_out, i_out):
    cand = jnp.concatenate([c_ref[j] for j in range(16)], axis=1)
    g = g_ref[...]
    lane = lax.broadcasted_iota(jnp.int32, (QT, GS), 1)
    idx = jnp.concatenate(
        [g[:, j:j + 1] * GS + lane for j in range(16)], axis=1)
    col = lax.broadcasted_iota(jnp.int32, (QT, 16 * GS), 1)
    cand = jnp.where((idx < n_keys) & (col < K_NN * GS), cand, NEG)
    svals, ivals = [], []
    for _ in range(K_NN):
        mx = jnp.max(cand, axis=1, keepdims=True)
        best = jnp.min(jnp.where(cand == mx, idx, BIG), axis=1,
                       keepdims=True)
        cand = jnp.where(idx == best, NEG, cand)
        svals.append(mx)
        ivals.append(best)
    s_out[...] = jnp.concatenate(svals, axis=1)
    i_out[...] = jnp.concatenate(ivals, axis=1).astype(jnp.int32)


def kernel(queries, keys):
    nq, d = queries.shape
    n_keys = keys.shape[0]
    ki_grid = -(-n_keys // KT)
    kp = ki_grid * KT
    n_groups = kp // GS
    mw = -(-ki_grid // 8) * 128
    qi_grid = nq // QT

    keys_p = jnp.concatenate(
        [keys, jnp.zeros((kp - n_keys, d), keys.dtype)], axis=0)

    s_full, m = pl.pallas_call(
        functools.partial(_p1_body, n_keys, ki_grid),
        grid=(qi_grid, ki_grid),
        in_specs=[
            pl.BlockSpec((QT, d), lambda qi, ki: (qi, 0)),
            pl.BlockSpec((KT, d), lambda qi, ki: (ki, 0)),
        ],
        out_specs=[
            pl.BlockSpec((GT, QT, GS), lambda qi, ki: (ki, qi, 0)),
            pl.BlockSpec((128, QT), lambda qi, ki: (ki // 8, qi)),
        ],
        out_shape=[
            jax.ShapeDtypeStruct((n_groups, nq, GS), jnp.float32),
            jax.ShapeDtypeStruct((mw, nq), jnp.float32),
        ],
    )(queries, keys_p)

    rowids_t, gq = pl.pallas_call(
        functools.partial(_p1b_body, nq),
        grid=(qi_grid,),
        in_specs=[pl.BlockSpec((mw, QT), lambda qi: (0, qi))],
        out_specs=[
            pl.BlockSpec((16, QT), lambda qi: (0, qi)),
            pl.BlockSpec((QT, 16), lambda qi: (qi, 0)),
        ],
        out_shape=[
            jax.ShapeDtypeStruct((16, nq), jnp.int32),
            jax.ShapeDtypeStruct((nq, 16), jnp.int32),
        ],
    )(m)

    n_rows = nq * 16
    chunks = n_rows // (32 * 128)
    mesh = plsc.VectorSubcoreMesh(core_axis_name="c", subcore_axis_name="s")
    gathered = pl.kernel(
        functools.partial(_sc_gather_body, chunks),
        mesh=mesh,
        out_type=jax.ShapeDtypeStruct((n_rows, GS), jnp.float32),
        scratch_types=[
            pltpu.VMEM((chunks, 128), jnp.int32),
            pltpu.VMEM((128, GS), jnp.float32),
            pltpu.SemaphoreType.DMA,
        ],
    )(s_full.reshape(n_groups * nq, GS),
      rowids_t.reshape(n_rows // 128, 128))

    return pl.pallas_call(
        functools.partial(_p3_body, n_keys),
        grid=(qi_grid,),
        in_specs=[
            pl.BlockSpec((16, QT, GS), lambda qi: (0, qi, 0)),
            pl.BlockSpec((QT, 16), lambda qi: (qi, 0)),
        ],
        out_specs=[
            pl.BlockSpec((QT, K_NN), lambda qi: (qi, 0)),
            pl.BlockSpec((QT, K_NN), lambda qi: (qi, 0)),
        ],
        out_shape=[
            jax.ShapeDtypeStruct((nq, K_NN), jnp.float32),
            jax.ShapeDtypeStruct((nq, K_NN), jnp.int32),
        ],
    )(gathered.reshape(16, nq, GS), gq)


# speedup vs baseline: 3.9886x; 1.2106x over previous
import functools

import jax
import jax.numpy as jnp
from jax import lax
from jax.experimental import pallas as pl
from jax.experimental.pallas import tpu as pltpu
from jax.experimental.pallas import tpu_sc as plsc

K_NN = 10
GS = 128
QT = 256
KT = 2048
GT = KT // GS
NEG = float("-inf")
BIG = 2**30


def _p1_body(n_keys, q_ref, k_ref, s_ref, m_ref):
    ki = pl.program_id(0)
    scores = lax.dot_general(
        q_ref[...], k_ref[...], (((1,), (1,)), ((), ())),
        preferred_element_type=jnp.float32)
    for j in range(GT):
        s_ref[j] = scores[:, j * GS:(j + 1) * GS]
    col_iota = lax.broadcasted_iota(jnp.int32, (QT, KT), 1)
    masked = jnp.where(col_iota + ki * KT < n_keys, scores, NEG)
    gmax = jnp.concatenate(
        [jnp.max(masked[:, j * GS:(j + 1) * GS], axis=1, keepdims=True)
         for j in range(GT)], axis=1)
    m_ref[...] = jnp.transpose(gmax)


def _p1b_body(nq, m_ref, r_ref, g_ref):
    qi = pl.program_id(0)
    m = m_ref[...]
    mh = m.shape[0]
    gids = lax.broadcasted_iota(jnp.int32, (mh, QT), 0)
    picks = []
    for _ in range(K_NN):
        mx = jnp.max(m, axis=0, keepdims=True)
        g = jnp.min(jnp.where(m == mx, gids, BIG), axis=0, keepdims=True)
        m = jnp.where(gids == g, NEG, m)
        picks.append(g)
    picks.extend([picks[-1]] * (16 - K_NN))
    gmat = jnp.concatenate(picks, axis=0)
    qlane = qi * QT + lax.broadcasted_iota(jnp.int32, (1, QT), 1)
    r_ref[...] = gmat * nq + qlane
    g_ref[...] = jnp.transpose(gmat)


def _sc_gather_body(chunks, table_hbm, idx_hbm, out_hbm, idx_v, rows_v, sem):
    wid = lax.axis_index("s") * 2 + lax.axis_index("c")
    pltpu.sync_copy(idx_hbm.at[pl.ds(wid * chunks, chunks)], idx_v)
    for c in range(chunks):
        pltpu.async_copy(table_hbm.at[idx_v.at[c]], rows_v, sem).wait()
        pltpu.sync_copy(rows_v,
                        out_hbm.at[pl.ds((wid * chunks + c) * 128, 128)])


def _p3_body(n_keys, c_ref, g_ref, s_out, i_out):
    cand = jnp.concatenate([c_ref[j] for j in range(16)], axis=1)
    g = g_ref[...]
    lane = lax.broadcasted_iota(jnp.int32, (QT, GS), 1)
    idx = jnp.concatenate(
        [g[:, j:j + 1] * GS + lane for j in range(16)], axis=1)
    col = lax.broadcasted_iota(jnp.int32, (QT, 16 * GS), 1)
    cand = jnp.where((idx < n_keys) & (col < K_NN * GS), cand, NEG)
    svals, ivals = [], []
    for _ in range(K_NN):
        mx = jnp.max(cand, axis=1, keepdims=True)
        best = jnp.min(jnp.where(cand == mx, idx, BIG), axis=1,
                       keepdims=True)
        cand = jnp.where(idx == best, NEG, cand)
        svals.append(mx)
        ivals.append(best)
    s_out[...] = jnp.concatenate(svals, axis=1)
    i_out[...] = jnp.concatenate(ivals, axis=1).astype(jnp.int32)


def kernel(queries, keys):
    nq, d = queries.shape
    n_keys = keys.shape[0]
    ki_grid = -(-n_keys // KT)
    kp = ki_grid * KT
    n_groups = kp // GS
    qi_grid = nq // QT

    s_full, m = pl.pallas_call(
        functools.partial(_p1_body, n_keys),
        grid=(ki_grid, qi_grid),
        in_specs=[
            pl.BlockSpec((QT, d), lambda ki, qi: (qi, 0)),
            pl.BlockSpec((KT, d), lambda ki, qi: (ki, 0)),
        ],
        out_specs=[
            pl.BlockSpec((GT, QT, GS), lambda ki, qi: (ki, qi, 0)),
            pl.BlockSpec((GT, QT), lambda ki, qi: (ki, qi)),
        ],
        out_shape=[
            jax.ShapeDtypeStruct((n_groups, nq, GS), jnp.float32),
            jax.ShapeDtypeStruct((n_groups, nq), jnp.float32),
        ],
    )(queries, keys)

    rowids_t, gq = pl.pallas_call(
        functools.partial(_p1b_body, nq),
        grid=(qi_grid,),
        in_specs=[pl.BlockSpec((n_groups, QT), lambda qi: (0, qi))],
        out_specs=[
            pl.BlockSpec((16, QT), lambda qi: (0, qi)),
            pl.BlockSpec((QT, 16), lambda qi: (qi, 0)),
        ],
        out_shape=[
            jax.ShapeDtypeStruct((16, nq), jnp.int32),
            jax.ShapeDtypeStruct((nq, 16), jnp.int32),
        ],
    )(m)

    n_rows = nq * 16
    chunks = n_rows // (32 * 128)
    mesh = plsc.VectorSubcoreMesh(core_axis_name="c", subcore_axis_name="s")
    gathered = pl.kernel(
        functools.partial(_sc_gather_body, chunks),
        mesh=mesh,
        out_type=jax.ShapeDtypeStruct((n_rows, GS), jnp.float32),
        scratch_types=[
            pltpu.VMEM((chunks, 128), jnp.int32),
            pltpu.VMEM((128, GS), jnp.float32),
            pltpu.SemaphoreType.DMA,
        ],
    )(s_full.reshape(n_groups * nq, GS),
      rowids_t.reshape(n_rows // 128, 128))

    return pl.pallas_call(
        functools.partial(_p3_body, n_keys),
        grid=(qi_grid,),
        in_specs=[
            pl.BlockSpec((16, QT, GS), lambda qi: (0, qi, 0)),
            pl.BlockSpec((QT, 16), lambda qi: (qi, 0)),
        ],
        out_specs=[
            pl.BlockSpec((QT, K_NN), lambda qi: (qi, 0)),
            pl.BlockSpec((QT, K_NN), lambda qi: (qi, 0)),
        ],
        out_shape=[
            jax.ShapeDtypeStruct((nq, K_NN), jnp.float32),
            jax.ShapeDtypeStruct((nq, K_NN), jnp.int32),
        ],
    )(gathered.reshape(16, nq, GS), gq)
